# baseline (speedup 1.0000x reference)
import functools

import jax
import jax.numpy as jnp
from jax import lax
from jax.experimental import pallas as pl
from jax.experimental.pallas import tpu as pltpu
from jax.experimental.pallas import tpu_sc as plsc

H = 4096
W = 4096
Q = 1048576

_info = plsc.get_sparse_core_info()
_NC, _NS, _L = _info.num_cores, _info.num_subcores, _info.num_lanes
_NW = _NC * _NS

_C = 512
_QPW = Q // _NW
_NCHUNK = _QPW // _C
_GL = 128
_NG = _C // _GL


def _sc_interp(q_hbm, grid_hbm, pars_hbm, out_hbm,
               qv, pv, itl, itr, ibl, ibr, vtl, vtr, vbl, vbr,
               axr, ayr, ov, sem):
    wid = lax.axis_index("s") * _NC + lax.axis_index("c")
    lanes = lax.iota(jnp.int32, _L)

    pltpu.sync_copy(pars_hbm, pv)
    oy = pv[pl.ds(0, _L)]
    dy = pv[pl.ds(_L, _L)]
    ox = pv[pl.ds(2 * _L, _L)]
    dx = pv[pl.ds(3 * _L, _L)]

    def chunk_body(c, _):
        base = wid * _QPW + c * _C
        pltpu.sync_copy(q_hbm.at[pl.ds(base * 2, _C * 2)], qv)

        def idx_body(j, _):
            ri = (j * _L + lanes) * 2
            y = plsc.load_gather(qv, [ri])
            x = plsc.load_gather(qv, [ri + 1])
            qyf = (jnp.float32(H) - 1.0) * (y - oy) / dy
            qxf = (jnp.float32(W) - 1.0) * (x - ox) / dx
            yi = jnp.minimum(jnp.maximum(qyf.astype(jnp.int32), 0), H - 2)
            xi = jnp.minimum(jnp.maximum(qxf.astype(jnp.int32), 0), W - 2)
            ay = jnp.minimum(jnp.maximum(qyf - yi.astype(jnp.float32), 0.0), 1.0)
            ax = jnp.minimum(jnp.maximum(qxf - xi.astype(jnp.float32), 0.0), 1.0)
            lin = yi * W + xi
            sl = pl.ds(j * _L, _L)
            itl[sl] = lin
            itr[sl] = lin + 1
            ibl[sl] = lin + W
            ibr[sl] = lin + (W + 1)
            axr[sl] = ax
            ayr[sl] = ay
            return ()

        lax.fori_loop(0, _C // _L, idx_body, (), unroll=False)

        cps = []
        for iref, vref in ((itl, vtl), (itr, vtr), (ibl, vbl), (ibr, vbr)):
            for g in range(_NG):
                sl = pl.ds(g * _GL, _GL)
                cps.append(pltpu.async_copy(grid_hbm.at[iref.at[sl]], vref.at[sl], sem))
        for cp in cps:
            cp.wait()

        def out_body(j, _):
            sl = pl.ds(j * _L, _L)
            tl = vtl[sl]
            tr = vtr[sl]
            bl = vbl[sl]
            br = vbr[sl]
            ax = axr[sl]
            ay = ayr[sl]
            top = ax * (tr - tl) + tl
            bot = ax * (br - bl) + bl
            ov[sl] = ay * (bot - top) + top
            return ()

        lax.fori_loop(0, _C // _L, out_body, (), unroll=False)
        pltpu.sync_copy(ov, out_hbm.at[pl.ds(base, _C)])
        return ()

    lax.fori_loop(0, _NCHUNK, chunk_body, (), unroll=False)


@jax.jit
def _run(q, gridf, pars):
    mesh = plsc.VectorSubcoreMesh(core_axis_name="c", subcore_axis_name="s")
    f = pl.kernel(
        _sc_interp,
        mesh=mesh,
        compiler_params=pltpu.CompilerParams(needs_layout_passes=False),
        out_type=jax.ShapeDtypeStruct((Q,), jnp.float32),
        scratch_types=[
            pltpu.VMEM((_C * 2,), jnp.float32),
            pltpu.VMEM((4 * _L,), jnp.float32),
            pltpu.VMEM((_C,), jnp.int32),
            pltpu.VMEM((_C,), jnp.int32),
            pltpu.VMEM((_C,), jnp.int32),
            pltpu.VMEM((_C,), jnp.int32),
            pltpu.VMEM((_C,), jnp.float32),
            pltpu.VMEM((_C,), jnp.float32),
            pltpu.VMEM((_C,), jnp.float32),
            pltpu.VMEM((_C,), jnp.float32),
            pltpu.VMEM((_C,), jnp.float32),
            pltpu.VMEM((_C,), jnp.float32),
            pltpu.VMEM((_C,), jnp.float32),
            pltpu.SemaphoreType.DMA,
        ],
    )
    return f(q, gridf, pars)


def kernel(inputs, grid, bounds):
    q = inputs[0].reshape(Q * 2)
    gridf = grid.reshape(H * W)
    oy = bounds[0, 0]
    dy = bounds[0, 1] - bounds[0, 0]
    ox = bounds[1, 0]
    dx = bounds[1, 1] - bounds[1, 0]
    pars = jnp.concatenate([
        jnp.full((_L,), oy, jnp.float32),
        jnp.full((_L,), dy, jnp.float32),
        jnp.full((_L,), ox, jnp.float32),
        jnp.full((_L,), dx, jnp.float32),
    ])
    out = _run(q, gridf, pars)
    return out[None, :, None]


# speedup vs baseline: 1.0781x; 1.0781x over previous
import functools

import jax
import jax.numpy as jnp
from jax import lax
from jax.experimental import pallas as pl
from jax.experimental.pallas import tpu as pltpu
from jax.experimental.pallas import tpu_sc as plsc

H = 4096
W = 4096
Q = 1048576

_info = plsc.get_sparse_core_info()
_NC, _NS, _L = _info.num_cores, _info.num_subcores, _info.num_lanes
_NW = _NC * _NS

_C = 512
_QPW = Q // _NW
_NCHUNK = _QPW // _C
_NPAIR = _NCHUNK // 2


def _sc_interp(q_hbm, grid_hbm, pars_hbm, out_hbm,
               pv,
               qv0, qv1, ix0, ix1, cv0, cv1, aw0, aw1, ov0, ov1,
               sq0, sq1, sg0, sg1, so0, so1):
    wid = lax.axis_index("s") * _NC + lax.axis_index("c")
    wbase = wid * _QPW
    lanes2 = lax.iota(jnp.int32, _L) * 2

    pltpu.sync_copy(pars_hbm, pv)
    oy = pv[pl.ds(0, _L)]
    sy = pv[pl.ds(_L, _L)]
    ox = pv[pl.ds(2 * _L, _L)]
    sx = pv[pl.ds(3 * _L, _L)]

    bufs = ((qv0, ix0, cv0, aw0, ov0, sq0, sg0, so0),
            (qv1, ix1, cv1, aw1, ov1, sq1, sg1, so1))

    def qslice(c):
        return q_hbm.at[pl.ds((wbase + c * _C) * 2, 2 * _C)]

    def fire_qload(c, b):
        qv, _, _, _, _, sq, _, _ = bufs[b]
        pltpu.async_copy(qslice(c), qv, sq)

    def compute_and_fire(c, b):
        qv, ix, cv, aw, _, sq, sg, _ = bufs[b]
        pltpu.make_async_copy(qslice(c), qv, sq).wait()

        @plsc.parallel_loop(0, _C // _L, unroll=2)
        def _(j):
            ri = j * (2 * _L) + lanes2
            y = plsc.load_gather(qv, [ri])
            x = plsc.load_gather(qv, [ri + 1])
            qyf = (y - oy) * sy
            qxf = (x - ox) * sx
            yi = jnp.minimum(jnp.maximum(qyf.astype(jnp.int32), 0), H - 2)
            xi = jnp.minimum(jnp.maximum(qxf.astype(jnp.int32), 0), W - 2)
            ay = jnp.minimum(jnp.maximum(qyf - yi.astype(jnp.float32), 0.0), 1.0)
            ax = jnp.minimum(jnp.maximum(qxf - xi.astype(jnp.float32), 0.0), 1.0)
            lin = lax.shift_left(yi, 12) + xi
            o = j * _L
            ix[pl.ds(o, _L)] = lin
            ix[pl.ds(_C + o, _L)] = lin + 1
            ix[pl.ds(2 * _C + o, _L)] = lin + W
            ix[pl.ds(3 * _C + o, _L)] = lin + (W + 1)
            aw[pl.ds(o, _L)] = ax
            aw[pl.ds(_C + o, _L)] = ay

        for k in range(4):
            sl = pl.ds(k * _C, _C)
            pltpu.async_copy(grid_hbm.at[ix.at[sl]], cv.at[sl], sg)

    def combine_and_store(c, b):
        _, ix, cv, aw, ov, _, sg, so = bufs[b]
        for k in range(4):
            sl = pl.ds(k * _C, _C)
            pltpu.make_async_copy(grid_hbm.at[ix.at[sl]], cv.at[sl], sg).wait()

        @pl.when(c >= 2)
        def _():
            pltpu.make_async_copy(ov, out_hbm.at[pl.ds(wbase, _C)], so).wait()

        @plsc.parallel_loop(0, _C // _L, unroll=2)
        def _(j):
            o = j * _L
            tl = cv[pl.ds(o, _L)]
            tr = cv[pl.ds(_C + o, _L)]
            bl = cv[pl.ds(2 * _C + o, _L)]
            br = cv[pl.ds(3 * _C + o, _L)]
            ax = aw[pl.ds(o, _L)]
            ay = aw[pl.ds(_C + o, _L)]
            top = ax * (tr - tl) + tl
            bot = ax * (br - bl) + bl
            ov[pl.ds(o, _L)] = ay * (bot - top) + top

        pltpu.async_copy(ov, out_hbm.at[pl.ds(wbase + c * _C, _C)], so)

    fire_qload(0, 0)
    compute_and_fire(0, 0)
    fire_qload(1, 1)

    def pair_body(g, _):
        c1 = 2 * g + 1
        compute_and_fire(c1, 1)

        @pl.when(c1 + 1 < _NCHUNK)
        def _():
            fire_qload(c1 + 1, 0)

        combine_and_store(2 * g, 0)

        @pl.when(c1 + 1 < _NCHUNK)
        def _():
            compute_and_fire(c1 + 1, 0)

        @pl.when(c1 + 2 < _NCHUNK)
        def _():
            fire_qload(c1 + 2, 1)

        combine_and_store(c1, 1)
        return ()

    lax.fori_loop(0, _NPAIR, pair_body, (), unroll=False)

    pltpu.make_async_copy(ov0, out_hbm.at[pl.ds(wbase, _C)], so0).wait()
    pltpu.make_async_copy(ov1, out_hbm.at[pl.ds(wbase, _C)], so1).wait()


@jax.jit
def _run(q, gridf, pars):
    mesh = plsc.VectorSubcoreMesh(core_axis_name="c", subcore_axis_name="s")
    f = pl.kernel(
        _sc_interp,
        mesh=mesh,
        compiler_params=pltpu.CompilerParams(needs_layout_passes=False),
        out_type=jax.ShapeDtypeStruct((Q,), jnp.float32),
        scratch_types=[
            pltpu.VMEM((4 * _L,), jnp.float32),
            pltpu.VMEM((_C * 2,), jnp.float32),
            pltpu.VMEM((_C * 2,), jnp.float32),
            pltpu.VMEM((4 * _C,), jnp.int32),
            pltpu.VMEM((4 * _C,), jnp.int32),
            pltpu.VMEM((4 * _C,), jnp.float32),
            pltpu.VMEM((4 * _C,), jnp.float32),
            pltpu.VMEM((2 * _C,), jnp.float32),
            pltpu.VMEM((2 * _C,), jnp.float32),
            pltpu.VMEM((_C,), jnp.float32),
            pltpu.VMEM((_C,), jnp.float32),
            pltpu.SemaphoreType.DMA,
            pltpu.SemaphoreType.DMA,
            pltpu.SemaphoreType.DMA,
            pltpu.SemaphoreType.DMA,
            pltpu.SemaphoreType.DMA,
            pltpu.SemaphoreType.DMA,
        ],
    )
    return f(q, gridf, pars)


def kernel(inputs, grid, bounds):
    q = inputs[0].reshape(Q * 2)
    gridf = grid.reshape(H * W)
    oy = bounds[0, 0]
    sy = (jnp.float32(H) - 1.0) / (bounds[0, 1] - bounds[0, 0])
    ox = bounds[1, 0]
    sx = (jnp.float32(W) - 1.0) / (bounds[1, 1] - bounds[1, 0])
    pars = jnp.concatenate([
        jnp.full((_L,), oy, jnp.float32),
        jnp.full((_L,), sy, jnp.float32),
        jnp.full((_L,), ox, jnp.float32),
        jnp.full((_L,), sx, jnp.float32),
    ])
    out = _run(q, gridf, pars)
    return out[None, :, None]


# speedup vs baseline: 3.5973x; 3.3368x over previous
import functools

import jax
import jax.numpy as jnp
from jax import lax
from jax.experimental import pallas as pl
from jax.experimental.pallas import tpu as pltpu
from jax.experimental.pallas import tpu_sc as plsc

H = 4096
W = 4096
Q = 1048576

_info = plsc.get_sparse_core_info()
_NC, _NS, _L = _info.num_cores, _info.num_subcores, _info.num_lanes
_NW = _NC * _NS

_C = 512
_QPW = Q // _NW
_NCHUNK = _QPW // _C
_NPAIR = _NCHUNK // 2


_TB = 8


def _detile_block(i_ref, o_ref):
    o_ref[...] = i_ref[...].reshape(_TB * W)


def _tc_detile(grid):
    return pl.pallas_call(
        _detile_block,
        grid=(H // _TB,),
        in_specs=[pl.BlockSpec((_TB, W), lambda i: (i, 0))],
        out_specs=pl.BlockSpec((_TB * W,), lambda i: (i,)),
        out_shape=jax.ShapeDtypeStruct((H * W,), jnp.float32),
    )(grid)


def _sc_interp(qy_hbm, qx_hbm, grid_hbm, pars_hbm, out_hbm,
               pv,
               qy0, qy1, qx0, qx1, ix0, ix1, cv0, cv1, aw0, aw1, ov0, ov1,
               sq0, sq1, sg0, sg1, so0, so1):
    wid = lax.axis_index("s") * _NC + lax.axis_index("c")
    wbase = wid * _QPW
    lanes2 = lax.iota(jnp.int32, _L) * 2

    pltpu.sync_copy(pars_hbm, pv)
    oy = pv[pl.ds(0, _L)]
    sy = pv[pl.ds(_L, _L)]
    ox = pv[pl.ds(2 * _L, _L)]
    sx = pv[pl.ds(3 * _L, _L)]

    bufs = ((qy0, qx0, ix0, cv0, aw0, ov0, sq0, sg0, so0),
            (qy1, qx1, ix1, cv1, aw1, ov1, sq1, sg1, so1))

    def yslice(c):
        return qy_hbm.at[pl.ds(wbase + c * _C, _C)]

    def xslice(c):
        return qx_hbm.at[pl.ds(wbase + c * _C, _C)]

    def fire_qload(c, b):
        qyv, qxv, _, _, _, _, sq, _, _ = bufs[b]
        pltpu.async_copy(yslice(c), qyv, sq)
        pltpu.async_copy(xslice(c), qxv, sq)

    def compute_and_fire(c, b):
        qyv, qxv, ix, cv, aw, _, sq, sg, _ = bufs[b]
        pltpu.make_async_copy(yslice(c), qyv, sq).wait()
        pltpu.make_async_copy(xslice(c), qxv, sq).wait()

        @plsc.parallel_loop(0, _C // _L, unroll=2)
        def _(j):
            o16 = j * _L
            y = qyv[pl.ds(o16, _L)]
            x = qxv[pl.ds(o16, _L)]
            qyf = (y - oy) * sy
            qxf = (x - ox) * sx
            yi = jnp.minimum(jnp.maximum(qyf.astype(jnp.int32), 0), H - 2)
            xi = jnp.minimum(jnp.maximum(qxf.astype(jnp.int32), 0), W - 2)
            ay = jnp.minimum(jnp.maximum(qyf - yi.astype(jnp.float32), 0.0), 1.0)
            ax = jnp.minimum(jnp.maximum(qxf - xi.astype(jnp.float32), 0.0), 1.0)
            lin = lax.shift_left(yi, 12) + xi
            o = j * _L
            pi = j * (2 * _L) + lanes2
            plsc.store_scatter(ix, [pi], lin)
            plsc.store_scatter(ix, [pi + 1], lin + 1)
            plsc.store_scatter(ix, [2 * _C + pi], lin + W)
            plsc.store_scatter(ix, [2 * _C + pi + 1], lin + (W + 1))
            aw[pl.ds(o, _L)] = ax
            aw[pl.ds(_C + o, _L)] = ay

        for k in range(4):
            sl = pl.ds(k * _C, _C)
            pltpu.async_copy(grid_hbm.at[ix.at[sl]], cv.at[sl], sg)

    def combine_and_store(c, b):
        _, _, ix, cv, aw, ov, _, sg, so = bufs[b]
        for k in range(4):
            sl = pl.ds(k * _C, _C)
            pltpu.make_async_copy(grid_hbm.at[ix.at[sl]], cv.at[sl], sg).wait()

        @pl.when(c >= 2)
        def _():
            pltpu.make_async_copy(ov, out_hbm.at[pl.ds(wbase, _C)], so).wait()

        @plsc.parallel_loop(0, _C // _L, unroll=2)
        def _(j):
            o = j * _L
            pi = j * (2 * _L) + lanes2
            tl = plsc.load_gather(cv, [pi])
            tr = plsc.load_gather(cv, [pi + 1])
            bl = plsc.load_gather(cv, [2 * _C + pi])
            br = plsc.load_gather(cv, [2 * _C + pi + 1])
            ax = aw[pl.ds(o, _L)]
            ay = aw[pl.ds(_C + o, _L)]
            top = ax * (tr - tl) + tl
            bot = ax * (br - bl) + bl
            ov[pl.ds(o, _L)] = ay * (bot - top) + top

        pltpu.async_copy(ov, out_hbm.at[pl.ds(wbase + c * _C, _C)], so)

    fire_qload(0, 0)
    compute_and_fire(0, 0)
    fire_qload(1, 1)

    def pair_body(g, _):
        c1 = 2 * g + 1
        compute_and_fire(c1, 1)

        @pl.when(c1 + 1 < _NCHUNK)
        def _():
            fire_qload(c1 + 1, 0)

        combine_and_store(2 * g, 0)

        @pl.when(c1 + 1 < _NCHUNK)
        def _():
            compute_and_fire(c1 + 1, 0)

        @pl.when(c1 + 2 < _NCHUNK)
        def _():
            fire_qload(c1 + 2, 1)

        combine_and_store(c1, 1)
        return ()

    lax.fori_loop(0, _NPAIR, pair_body, (), unroll=False)

    pltpu.make_async_copy(ov0, out_hbm.at[pl.ds(wbase, _C)], so0).wait()
    pltpu.make_async_copy(ov1, out_hbm.at[pl.ds(wbase, _C)], so1).wait()


@jax.jit
def _run(qy, qx, gridf, pars):
    mesh = plsc.VectorSubcoreMesh(core_axis_name="c", subcore_axis_name="s")
    f = pl.kernel(
        _sc_interp,
        mesh=mesh,
        compiler_params=pltpu.CompilerParams(needs_layout_passes=False),
        out_type=jax.ShapeDtypeStruct((Q,), jnp.float32),
        scratch_types=[
            pltpu.VMEM((4 * _L,), jnp.float32),
            pltpu.VMEM((_C,), jnp.float32),
            pltpu.VMEM((_C,), jnp.float32),
            pltpu.VMEM((_C,), jnp.float32),
            pltpu.VMEM((_C,), jnp.float32),
            pltpu.VMEM((4 * _C,), jnp.int32),
            pltpu.VMEM((4 * _C,), jnp.int32),
            pltpu.VMEM((4 * _C,), jnp.float32),
            pltpu.VMEM((4 * _C,), jnp.float32),
            pltpu.VMEM((2 * _C,), jnp.float32),
            pltpu.VMEM((2 * _C,), jnp.float32),
            pltpu.VMEM((_C,), jnp.float32),
            pltpu.VMEM((_C,), jnp.float32),
            pltpu.SemaphoreType.DMA,
            pltpu.SemaphoreType.DMA,
            pltpu.SemaphoreType.DMA,
            pltpu.SemaphoreType.DMA,
            pltpu.SemaphoreType.DMA,
            pltpu.SemaphoreType.DMA,
        ],
    )
    return f(qy, qx, gridf, pars)


def kernel(inputs, grid, bounds):
    one = (bounds[0, 1] - bounds[0, 0]) / (bounds[0, 1] - bounds[0, 0])
    qy = inputs[0, :, 0] * one
    qx = inputs[0, :, 1] * one
    gridf = _tc_detile(grid)
    oy = bounds[0, 0]
    sy = (jnp.float32(H) - 1.0) / (bounds[0, 1] - bounds[0, 0])
    ox = bounds[1, 0]
    sx = (jnp.float32(W) - 1.0) / (bounds[1, 1] - bounds[1, 0])
    pars = jnp.concatenate([
        jnp.full((_L,), oy, jnp.float32),
        jnp.full((_L,), sy, jnp.float32),
        jnp.full((_L,), ox, jnp.float32),
        jnp.full((_L,), sx, jnp.float32),
    ])
    out = _run(qy, qx, gridf, pars)
    return out[None, :, None]


# speedup vs baseline: 6.5389x; 1.8177x over previous
import functools

import jax
import jax.numpy as jnp
from jax import lax
from jax.experimental import pallas as pl
from jax.experimental.pallas import tpu as pltpu
from jax.experimental.pallas import tpu_sc as plsc

H = 4096
W = 4096
Q = 1048576

_info = plsc.get_sparse_core_info()
_NC, _NS, _L = _info.num_cores, _info.num_subcores, _info.num_lanes
_NW = _NC * _NS

_C = 1024
_QPW = Q // _NW
_NCHUNK = _QPW // _C
_NPAIR = _NCHUNK // 2


_TB = 64


def _detile_block(i_ref, o_ref):
    o_ref[...] = i_ref[...].reshape(_TB * W)


def _tc_detile(grid):
    return pl.pallas_call(
        _detile_block,
        grid=(H // _TB,),
        in_specs=[pl.BlockSpec((_TB, W), lambda i: (i, 0))],
        out_specs=pl.BlockSpec((_TB * W,), lambda i: (i,)),
        out_shape=jax.ShapeDtypeStruct((H * W,), jnp.float32),
    )(grid)


def _sc_interp(qy_hbm, qx_hbm, grid_hbm, pars_hbm, out_hbm,
               pv,
               qy0, qy1, qx0, qx1, ix0, ix1, cv0, cv1, aw0, aw1, ov0, ov1,
               sq0, sq1, sg0, sg1, so0, so1):
    wid = lax.axis_index("s") * _NC + lax.axis_index("c")
    wbase = wid * _QPW
    lanes2 = lax.iota(jnp.int32, _L) * 2

    pltpu.sync_copy(pars_hbm, pv)
    oy = pv[pl.ds(0, _L)]
    sy = pv[pl.ds(_L, _L)]
    ox = pv[pl.ds(2 * _L, _L)]
    sx = pv[pl.ds(3 * _L, _L)]

    bufs = ((qy0, qx0, ix0, cv0, aw0, ov0, sq0, sg0, so0),
            (qy1, qx1, ix1, cv1, aw1, ov1, sq1, sg1, so1))

    def yslice(c):
        return qy_hbm.at[pl.ds(wbase + c * _C, _C)]

    def xslice(c):
        return qx_hbm.at[pl.ds(wbase + c * _C, _C)]

    def fire_qload(c, b):
        qyv, qxv, _, _, _, _, sq, _, _ = bufs[b]
        pltpu.async_copy(yslice(c), qyv, sq)
        pltpu.async_copy(xslice(c), qxv, sq)

    def compute_and_fire(c, b):
        qyv, qxv, ix, cv, aw, _, sq, sg, _ = bufs[b]
        pltpu.make_async_copy(yslice(c), qyv, sq).wait()
        pltpu.make_async_copy(xslice(c), qxv, sq).wait()

        @plsc.parallel_loop(0, _C // _L, unroll=4)
        def _(j):
            o16 = j * _L
            y = qyv[pl.ds(o16, _L)]
            x = qxv[pl.ds(o16, _L)]
            qyf = (y - oy) * sy
            qxf = (x - ox) * sx
            yi = jnp.minimum(jnp.maximum(qyf.astype(jnp.int32), 0), H - 2)
            xi = jnp.minimum(jnp.maximum(qxf.astype(jnp.int32), 0), W - 2)
            ay = jnp.minimum(jnp.maximum(qyf - yi.astype(jnp.float32), 0.0), 1.0)
            ax = jnp.minimum(jnp.maximum(qxf - xi.astype(jnp.float32), 0.0), 1.0)
            lin = lax.shift_left(yi, 12) + xi
            o = j * _L
            pi = j * (2 * _L) + lanes2
            plsc.store_scatter(ix, [pi], lin)
            plsc.store_scatter(ix, [pi + 1], lin + 1)
            plsc.store_scatter(ix, [2 * _C + pi], lin + W)
            plsc.store_scatter(ix, [2 * _C + pi + 1], lin + (W + 1))
            aw[pl.ds(o, _L)] = ax
            aw[pl.ds(_C + o, _L)] = ay

        for k in range(4):
            sl = pl.ds(k * _C, _C)
            pltpu.async_copy(grid_hbm.at[ix.at[sl]], cv.at[sl], sg)

    def combine_and_store(c, b):
        _, _, ix, cv, aw, ov, _, sg, so = bufs[b]
        for k in range(4):
            sl = pl.ds(k * _C, _C)
            pltpu.make_async_copy(grid_hbm.at[ix.at[sl]], cv.at[sl], sg).wait()

        @pl.when(c >= 2)
        def _():
            pltpu.make_async_copy(ov, out_hbm.at[pl.ds(wbase, _C)], so).wait()

        @plsc.parallel_loop(0, _C // _L, unroll=4)
        def _(j):
            o = j * _L
            pi = j * (2 * _L) + lanes2
            tl = plsc.load_gather(cv, [pi])
            tr = plsc.load_gather(cv, [pi + 1])
            bl = plsc.load_gather(cv, [2 * _C + pi])
            br = plsc.load_gather(cv, [2 * _C + pi + 1])
            ax = aw[pl.ds(o, _L)]
            ay = aw[pl.ds(_C + o, _L)]
            top = ax * (tr - tl) + tl
            bot = ax * (br - bl) + bl
            ov[pl.ds(o, _L)] = ay * (bot - top) + top

        pltpu.async_copy(ov, out_hbm.at[pl.ds(wbase + c * _C, _C)], so)

    fire_qload(0, 0)
    compute_and_fire(0, 0)
    fire_qload(1, 1)

    def pair_body(g, _):
        c1 = 2 * g + 1
        compute_and_fire(c1, 1)

        @pl.when(c1 + 1 < _NCHUNK)
        def _():
            fire_qload(c1 + 1, 0)

        combine_and_store(2 * g, 0)

        @pl.when(c1 + 1 < _NCHUNK)
        def _():
            compute_and_fire(c1 + 1, 0)

        @pl.when(c1 + 2 < _NCHUNK)
        def _():
            fire_qload(c1 + 2, 1)

        combine_and_store(c1, 1)
        return ()

    lax.fori_loop(0, _NPAIR, pair_body, (), unroll=False)

    pltpu.make_async_copy(ov0, out_hbm.at[pl.ds(wbase, _C)], so0).wait()
    pltpu.make_async_copy(ov1, out_hbm.at[pl.ds(wbase, _C)], so1).wait()


@jax.jit
def _run(qy, qx, gridf, pars):
    mesh = plsc.VectorSubcoreMesh(core_axis_name="c", subcore_axis_name="s")
    f = pl.kernel(
        _sc_interp,
        mesh=mesh,
        compiler_params=pltpu.CompilerParams(needs_layout_passes=False),
        out_type=jax.ShapeDtypeStruct((Q,), jnp.float32),
        scratch_types=[
            pltpu.VMEM((4 * _L,), jnp.float32),
            pltpu.VMEM((_C,), jnp.float32),
            pltpu.VMEM((_C,), jnp.float32),
            pltpu.VMEM((_C,), jnp.float32),
            pltpu.VMEM((_C,), jnp.float32),
            pltpu.VMEM((4 * _C,), jnp.int32),
            pltpu.VMEM((4 * _C,), jnp.int32),
            pltpu.VMEM((4 * _C,), jnp.float32),
            pltpu.VMEM((4 * _C,), jnp.float32),
            pltpu.VMEM((2 * _C,), jnp.float32),
            pltpu.VMEM((2 * _C,), jnp.float32),
            pltpu.VMEM((_C,), jnp.float32),
            pltpu.VMEM((_C,), jnp.float32),
            pltpu.SemaphoreType.DMA,
            pltpu.SemaphoreType.DMA,
            pltpu.SemaphoreType.DMA,
            pltpu.SemaphoreType.DMA,
            pltpu.SemaphoreType.DMA,
            pltpu.SemaphoreType.DMA,
        ],
    )
    return f(qy, qx, gridf, pars)


def kernel(inputs, grid, bounds):
    one = (bounds[0, 1] - bounds[0, 0]) / (bounds[0, 1] - bounds[0, 0])
    qy = inputs[0, :, 0] * one
    qx = inputs[0, :, 1] * one
    gridf = _tc_detile(grid)
    oy = bounds[0, 0]
    sy = (jnp.float32(H) - 1.0) / (bounds[0, 1] - bounds[0, 0])
    ox = bounds[1, 0]
    sx = (jnp.float32(W) - 1.0) / (bounds[1, 1] - bounds[1, 0])
    pars = jnp.concatenate([
        jnp.full((_L,), oy, jnp.float32),
        jnp.full((_L,), sy, jnp.float32),
        jnp.full((_L,), ox, jnp.float32),
        jnp.full((_L,), sx, jnp.float32),
    ])
    out = _run(qy, qx, gridf, pars)
    return out[None, :, None]


# speedup vs baseline: 7.1898x; 1.0996x over previous
import functools

import jax
import jax.numpy as jnp
from jax import lax
from jax.experimental import pallas as pl
from jax.experimental.pallas import tpu as pltpu
from jax.experimental.pallas import tpu_sc as plsc

H = 4096
W = 4096
Q = 1048576

_info = plsc.get_sparse_core_info()
_NC, _NS, _L = _info.num_cores, _info.num_subcores, _info.num_lanes
_NW = _NC * _NS

_C = 2048
_QPW = Q // _NW
_NCHUNK = _QPW // _C
_NPAIR = _NCHUNK // 2


_TB = 256


def _detile_block(i_ref, o_ref):
    o_ref[...] = i_ref[...].reshape(_TB * W)


def _tc_detile(grid):
    return pl.pallas_call(
        _detile_block,
        grid=(H // _TB,),
        in_specs=[pl.BlockSpec((_TB, W), lambda i: (i, 0))],
        out_specs=pl.BlockSpec((_TB * W,), lambda i: (i,)),
        out_shape=jax.ShapeDtypeStruct((H * W,), jnp.float32),
    )(grid)


def _sc_interp(qy_hbm, qx_hbm, grid_hbm, pars_hbm, out_hbm,
               pv,
               qy0, qy1, qx0, qx1, ix0, ix1, cv0, cv1, aw0, aw1, ov0, ov1,
               sq0, sq1, sg0, sg1, so0, so1):
    wid = lax.axis_index("s") * _NC + lax.axis_index("c")
    wbase = wid * _QPW
    lanes2 = lax.iota(jnp.int32, _L) * 2

    pltpu.sync_copy(pars_hbm, pv)
    oy = pv[pl.ds(0, _L)]
    sy = pv[pl.ds(_L, _L)]
    ox = pv[pl.ds(2 * _L, _L)]
    sx = pv[pl.ds(3 * _L, _L)]

    bufs = ((qy0, qx0, ix0, cv0, aw0, ov0, sq0, sg0, so0),
            (qy1, qx1, ix1, cv1, aw1, ov1, sq1, sg1, so1))

    def yslice(c):
        return qy_hbm.at[pl.ds(wbase + c * _C, _C)]

    def xslice(c):
        return qx_hbm.at[pl.ds(wbase + c * _C, _C)]

    def fire_qload(c, b):
        qyv, qxv, _, _, _, _, sq, _, _ = bufs[b]
        pltpu.async_copy(yslice(c), qyv, sq)
        pltpu.async_copy(xslice(c), qxv, sq)

    def compute_and_fire(c, b):
        qyv, qxv, ix, cv, aw, _, sq, sg, _ = bufs[b]
        pltpu.make_async_copy(yslice(c), qyv, sq).wait()
        pltpu.make_async_copy(xslice(c), qxv, sq).wait()

        @plsc.parallel_loop(0, _C // _L, unroll=4)
        def _(j):
            o16 = j * _L
            y = qyv[pl.ds(o16, _L)]
            x = qxv[pl.ds(o16, _L)]
            qyf = (y - oy) * sy
            qxf = (x - ox) * sx
            yi = jnp.minimum(jnp.maximum(qyf.astype(jnp.int32), 0), H - 2)
            xi = jnp.minimum(jnp.maximum(qxf.astype(jnp.int32), 0), W - 2)
            ay = jnp.minimum(jnp.maximum(qyf - yi.astype(jnp.float32), 0.0), 1.0)
            ax = jnp.minimum(jnp.maximum(qxf - xi.astype(jnp.float32), 0.0), 1.0)
            lin = lax.shift_left(yi, 12) + xi
            o = j * _L
            pi = j * (2 * _L) + lanes2
            plsc.store_scatter(ix, [pi], lin)
            plsc.store_scatter(ix, [pi + 1], lin + 1)
            plsc.store_scatter(ix, [2 * _C + pi], lin + W)
            plsc.store_scatter(ix, [2 * _C + pi + 1], lin + (W + 1))
            aw[pl.ds(o, _L)] = ax
            aw[pl.ds(_C + o, _L)] = ay

        for k in range(4):
            sl = pl.ds(k * _C, _C)
            pltpu.async_copy(grid_hbm.at[ix.at[sl]], cv.at[sl], sg)

    def combine_and_store(c, b):
        _, _, ix, cv, aw, ov, _, sg, so = bufs[b]
        for k in range(4):
            sl = pl.ds(k * _C, _C)
            pltpu.make_async_copy(grid_hbm.at[ix.at[sl]], cv.at[sl], sg).wait()

        @pl.when(c >= 2)
        def _():
            pltpu.make_async_copy(ov, out_hbm.at[pl.ds(wbase, _C)], so).wait()

        @plsc.parallel_loop(0, _C // _L, unroll=4)
        def _(j):
            o = j * _L
            pi = j * (2 * _L) + lanes2
            tl = plsc.load_gather(cv, [pi])
            tr = plsc.load_gather(cv, [pi + 1])
            bl = plsc.load_gather(cv, [2 * _C + pi])
            br = plsc.load_gather(cv, [2 * _C + pi + 1])
            ax = aw[pl.ds(o, _L)]
            ay = aw[pl.ds(_C + o, _L)]
            top = ax * (tr - tl) + tl
            bot = ax * (br - bl) + bl
            ov[pl.ds(o, _L)] = ay * (bot - top) + top

        pltpu.async_copy(ov, out_hbm.at[pl.ds(wbase + c * _C, _C)], so)

    fire_qload(0, 0)
    compute_and_fire(0, 0)
    fire_qload(1, 1)

    def pair_body(g, _):
        c1 = 2 * g + 1
        compute_and_fire(c1, 1)

        @pl.when(c1 + 1 < _NCHUNK)
        def _():
            fire_qload(c1 + 1, 0)

        combine_and_store(2 * g, 0)

        @pl.when(c1 + 1 < _NCHUNK)
        def _():
            compute_and_fire(c1 + 1, 0)

        @pl.when(c1 + 2 < _NCHUNK)
        def _():
            fire_qload(c1 + 2, 1)

        combine_and_store(c1, 1)
        return ()

    lax.fori_loop(0, _NPAIR, pair_body, (), unroll=False)

    pltpu.make_async_copy(ov0, out_hbm.at[pl.ds(wbase, _C)], so0).wait()
    pltpu.make_async_copy(ov1, out_hbm.at[pl.ds(wbase, _C)], so1).wait()


@jax.jit
def _run(qy, qx, gridf, pars):
    mesh = plsc.VectorSubcoreMesh(core_axis_name="c", subcore_axis_name="s")
    f = pl.kernel(
        _sc_interp,
        mesh=mesh,
        compiler_params=pltpu.CompilerParams(needs_layout_passes=False),
        out_type=jax.ShapeDtypeStruct((Q,), jnp.float32),
        scratch_types=[
            pltpu.VMEM((4 * _L,), jnp.float32),
            pltpu.VMEM((_C,), jnp.float32),
            pltpu.VMEM((_C,), jnp.float32),
            pltpu.VMEM((_C,), jnp.float32),
            pltpu.VMEM((_C,), jnp.float32),
            pltpu.VMEM((4 * _C,), jnp.int32),
            pltpu.VMEM((4 * _C,), jnp.int32),
            pltpu.VMEM((4 * _C,), jnp.float32),
            pltpu.VMEM((4 * _C,), jnp.float32),
            pltpu.VMEM((2 * _C,), jnp.float32),
            pltpu.VMEM((2 * _C,), jnp.float32),
            pltpu.VMEM((_C,), jnp.float32),
            pltpu.VMEM((_C,), jnp.float32),
            pltpu.SemaphoreType.DMA,
            pltpu.SemaphoreType.DMA,
            pltpu.SemaphoreType.DMA,
            pltpu.SemaphoreType.DMA,
            pltpu.SemaphoreType.DMA,
            pltpu.SemaphoreType.DMA,
        ],
    )
    return f(qy, qx, gridf, pars)


def kernel(inputs, grid, bounds):
    one = (bounds[0, 1] - bounds[0, 0]) / (bounds[0, 1] - bounds[0, 0])
    qy = inputs[0, :, 0] * one
    qx = inputs[0, :, 1] * one
    gridf = _tc_detile(grid)
    oy = bounds[0, 0]
    sy = (jnp.float32(H) - 1.0) / (bounds[0, 1] - bounds[0, 0])
    ox = bounds[1, 0]
    sx = (jnp.float32(W) - 1.0) / (bounds[1, 1] - bounds[1, 0])
    pars = jnp.concatenate([
        jnp.full((_L,), oy, jnp.float32),
        jnp.full((_L,), sy, jnp.float32),
        jnp.full((_L,), ox, jnp.float32),
        jnp.full((_L,), sx, jnp.float32),
    ])
    out = _run(qy, qx, gridf, pars)
    return out[None, :, None]
